# baseline (speedup 1.0000x reference)
import functools

import jax
import jax.numpy as jnp
from jax import lax
from jax.experimental import pallas as pl
from jax.experimental.pallas import tpu as pltpu
from jax.experimental.pallas import tpu_sc as plsc

B = 4096
L = 200
D = 32
PAD = 0


_NC = 2
_NS = 16
_NW = _NC * _NS

_TOTAL = B * L
_PER_W = _TOTAL // _NW
_CHUNK = 1024
_N_CHUNKS = _PER_W // _CHUNK
_IDXW = 128
_GPC = _CHUNK // _IDXW


_BBLK = B // _CHUNK
_TASKS = L * _BBLK
_TASKS_PER_W = _TASKS // _NW

V = 1000000
_UNITS = V // 128
_REM = V - _UNITS * 128
_PR = _UNITS * 32 + 32


def _sc_wprep(wt, wrem):
    mesh = plsc.VectorSubcoreMesh(core_axis_name="c", subcore_axis_name="s")

    @functools.partial(
        pl.kernel,
        mesh=mesh,
        out_type=jax.ShapeDtypeStruct((_PR, 128), jnp.float32),
        scratch_types=[
            pltpu.VMEM((D, 128), jnp.float32),
            pltpu.VMEM((32, 128), jnp.float32),
        ],
        compiler_params=pltpu.CompilerParams(
            use_tc_tiling_on_sc=True, needs_layout_passes=False
        ),
    )
    def k(wt_hbm, wrem_hbm, out_hbm, src_v, dst_v):
        wid = lax.axis_index("s") * _NC + lax.axis_index("c")
        n_units = 244 + jnp.where(wid < _UNITS - 244 * _NW, 1, 0)
        lanes = lax.iota(jnp.int32, 16)

        def body(i, carry):
            u = wid + i * _NW
            c0 = pl.multiple_of(u * 128, 128)
            pltpu.sync_copy(wt_hbm.at[:, pl.ds(c0, 128)], src_v)
            for q in range(32):
                for g in range(8):
                    kk, d0 = g >> 1, (g & 1) * 16
                    vals = plsc.load_gather(
                        src_v, [lanes + d0, jnp.full((16,), 4 * q + kk, jnp.int32)]
                    )
                    dst_v[q, pl.ds(g * 16, 16)] = vals
            r0 = pl.multiple_of(u * 32, 32)
            pltpu.sync_copy(dst_v, out_hbm.at[pl.ds(r0, 32)])
            return carry

        lax.fori_loop(0, n_units, body, 0)

        @pl.when(wid == _NW - 1)
        def _():
            pltpu.sync_copy(wrem_hbm, dst_v.at[pl.ds(0, 16)])
            pltpu.sync_copy(dst_v.at[pl.ds(0, 16)],
                            out_hbm.at[pl.ds(_UNITS * 32, 16)])

    return k(wt, wrem)


def _sc_gather(idx2d, table):
    mesh = plsc.VectorSubcoreMesh(core_axis_name="c", subcore_axis_name="s")

    @functools.partial(
        pl.kernel,
        mesh=mesh,
        out_type=jax.ShapeDtypeStruct((L, B, D), jnp.float32),
        scratch_types=[
            pltpu.VMEM((_GPC, _IDXW), jnp.int32),
            pltpu.VMEM((_CHUNK, D), jnp.float32),
            pltpu.SemaphoreType.DMA,
        ],
        compiler_params=pltpu.CompilerParams(use_tc_tiling_on_sc=False),
    )
    def k(idx_hbm, w_hbm, out_hbm, idx_v, rows_v, sem):
        wid = lax.axis_index("s") * _NC + lax.axis_index("c")

        def body(i, carry):
            t = wid * _TASKS_PER_W + i
            l = t // _BBLK
            bb = t % _BBLK
            roff = l * (B // _IDXW) + bb * _GPC
            pltpu.sync_copy(idx_hbm.at[pl.ds(roff, _GPC)], idx_v)
            for j in range(_GPC):
                pltpu.async_copy(
                    w_hbm.at[idx_v.at[j]],
                    rows_v.at[pl.ds(j * _IDXW, _IDXW)],
                    sem,
                )
            for j in range(_GPC):
                pltpu.make_async_copy(
                    w_hbm.at[idx_v.at[j]],
                    rows_v.at[pl.ds(j * _IDXW, _IDXW)],
                    sem,
                ).wait()
            pltpu.sync_copy(rows_v, out_hbm.at[l, pl.ds(bb * _CHUNK, _CHUNK)])
            return carry

        lax.fori_loop(0, _TASKS_PER_W, body, 0)

    return k(idx2d, table)


def _mask_body(wt_ref, out_ref):
    i = pl.program_id(0)
    nz = wt_ref[...] != PAD
    jj = lax.broadcasted_iota(jnp.int32, (L, B), 0)
    out_ref[...] = ((jj <= i) & nz).astype(jnp.int8)[None]


def _mask_t8(words_t):
    return pl.pallas_call(
        _mask_body,
        grid=(L,),
        in_specs=[pl.BlockSpec((L, B), lambda i: (0, 0))],
        out_specs=pl.BlockSpec((1, L, B), lambda i: (i, 0, 0)),
        out_shape=jax.ShapeDtypeStruct((L, L, B), jnp.int8),
    )(words_t)


def kernel(batch_words, W):
    words_t = batch_words.T
    idx2d = words_t.reshape(_TOTAL // _IDXW, _IDXW)
    wrem = W[_UNITS * 128:, :].reshape(16, 128)
    w_lin = _sc_wprep(W.T, wrem).reshape(_PR * 4, D)
    emb = _sc_gather(idx2d, w_lin).transpose(1, 0, 2)
    mask_t = _mask_t8(words_t) != 0
    masks = jnp.transpose(mask_t, (2, 0, 1))
    return emb, masks


# speedup vs baseline: 1.7443x; 1.7443x over previous
import functools

import jax
import jax.numpy as jnp
from jax import lax
from jax.experimental import pallas as pl
from jax.experimental.pallas import tpu as pltpu
from jax.experimental.pallas import tpu_sc as plsc

B = 4096
L = 200
D = 32
PAD = 0


_NC = 2
_NS = 16
_NW = _NC * _NS

_TOTAL = B * L
_PER_W = _TOTAL // _NW
_CHUNK = 1024
_N_CHUNKS = _PER_W // _CHUNK
_IDXW = 128
_GPC = _CHUNK // _IDXW


_BBLK = B // _CHUNK
_TASKS = L * _BBLK
_TASKS_PER_W = _TASKS // _NW

V = 1000000
_SEG = 1 << 18
_WBLK = 1024


def _wprep_body(w0, w1, w2, w3, out_ref):
    out_ref[:, 0:32] = w0[...].T
    out_ref[:, 32:64] = w1[...].T
    out_ref[:, 64:96] = w2[...].T
    out_ref[:, 96:128] = w3[...].T


def _tc_wprep(wt):
    nseg_blocks = _SEG // _WBLK
    max_blk = V // _WBLK
    specs = [
        pl.BlockSpec(
            (D, _WBLK),
            (lambda i, k=k: (0, jnp.minimum(k * nseg_blocks + i, max_blk))),
        )
        for k in range(4)
    ]
    return pl.pallas_call(
        _wprep_body,
        grid=(nseg_blocks,),
        in_specs=specs,
        out_specs=pl.BlockSpec((_WBLK, 128), lambda i: (i, 0)),
        out_shape=jax.ShapeDtypeStruct((_SEG, 128), jnp.float32),
    )(wt, wt, wt, wt)


def _sc_gather(idx2d, table):
    mesh = plsc.VectorSubcoreMesh(core_axis_name="c", subcore_axis_name="s")

    @functools.partial(
        pl.kernel,
        mesh=mesh,
        out_type=jax.ShapeDtypeStruct((L, B, D), jnp.float32),
        scratch_types=[
            pltpu.VMEM((_GPC, _IDXW), jnp.int32),
            pltpu.VMEM((_GPC, _IDXW), jnp.int32),
            pltpu.VMEM((_CHUNK, D), jnp.float32),
            pltpu.SemaphoreType.DMA,
        ],
        compiler_params=pltpu.CompilerParams(use_tc_tiling_on_sc=False),
    )
    def k(idx_hbm, w_hbm, out_hbm, idx_v, idx_r, rows_v, sem):
        wid = lax.axis_index("s") * _NC + lax.axis_index("c")

        def body(i, carry):
            t = wid * _TASKS_PER_W + i
            l = t // _BBLK
            bb = t % _BBLK
            roff = l * (B // _IDXW) + bb * _GPC
            pltpu.sync_copy(idx_hbm.at[pl.ds(roff, _GPC)], idx_v)
            for j in range(_GPC):
                for c0 in range(0, _IDXW, 16):
                    v = idx_v[j, pl.ds(c0, 16)]
                    idx_r[j, pl.ds(c0, 16)] = (
                        ((v & (_SEG - 1)) << 2) | jax.lax.shift_right_logical(v, 18)
                    )
            for j in range(_GPC):
                pltpu.async_copy(
                    w_hbm.at[idx_r.at[j]],
                    rows_v.at[pl.ds(j * _IDXW, _IDXW)],
                    sem,
                )
            for j in range(_GPC):
                pltpu.make_async_copy(
                    w_hbm.at[idx_r.at[j]],
                    rows_v.at[pl.ds(j * _IDXW, _IDXW)],
                    sem,
                ).wait()
            pltpu.sync_copy(rows_v, out_hbm.at[l, pl.ds(bb * _CHUNK, _CHUNK)])
            return carry

        lax.fori_loop(0, _TASKS_PER_W, body, 0)

    return k(idx2d, table)


def _mask_body(wt_ref, out_ref):
    i = pl.program_id(0)
    nz = wt_ref[...] != PAD
    jj = lax.broadcasted_iota(jnp.int32, (L, B), 0)
    out_ref[...] = ((jj <= i) & nz).astype(jnp.int8)[None]


def _mask_t8(words_t):
    return pl.pallas_call(
        _mask_body,
        grid=(L,),
        in_specs=[pl.BlockSpec((L, B), lambda i: (0, 0))],
        out_specs=pl.BlockSpec((1, L, B), lambda i: (i, 0, 0)),
        out_shape=jax.ShapeDtypeStruct((L, L, B), jnp.int8),
    )(words_t)


def kernel(batch_words, W):
    words_t = batch_words.T
    idx2d = words_t.reshape(_TOTAL // _IDXW, _IDXW)
    w_lin = _tc_wprep(W.T).reshape(4 * _SEG, D)
    emb = _sc_gather(idx2d, w_lin).transpose(1, 0, 2)
    mask_t = _mask_t8(words_t) != 0
    masks = jnp.transpose(mask_t, (2, 0, 1))
    return emb, masks


# speedup vs baseline: 1.8409x; 1.0554x over previous
import functools

import jax
import jax.numpy as jnp
from jax import lax
from jax.experimental import pallas as pl
from jax.experimental.pallas import tpu as pltpu
from jax.experimental.pallas import tpu_sc as plsc

B = 4096
L = 200
D = 32
PAD = 0


_NC = 2
_NS = 16
_NW = _NC * _NS

_TOTAL = B * L
_PER_W = _TOTAL // _NW
_CHUNK = 1024
_N_CHUNKS = _PER_W // _CHUNK
_IDXW = 128
_GPC = _CHUNK // _IDXW


_BBLK = B // _CHUNK
_TASKS = L * _BBLK
_TASKS_PER_W = _TASKS // _NW

V = 1000000
_SEG = 1 << 18
_WBLK = 1024


def _wprep_body(w0, w1, w2, w3, out_ref):
    out_ref[:, 0:32] = w0[...].T
    out_ref[:, 32:64] = w1[...].T
    out_ref[:, 64:96] = w2[...].T
    out_ref[:, 96:128] = w3[...].T


def _tc_wprep(wt):
    nseg_blocks = _SEG // _WBLK
    max_blk = V // _WBLK
    specs = [
        pl.BlockSpec(
            (D, _WBLK),
            (lambda i, k=k: (0, jnp.minimum(k * nseg_blocks + i, max_blk))),
        )
        for k in range(4)
    ]
    return pl.pallas_call(
        _wprep_body,
        grid=(nseg_blocks,),
        in_specs=specs,
        out_specs=pl.BlockSpec((_WBLK, 128), lambda i: (i, 0)),
        out_shape=jax.ShapeDtypeStruct((_SEG, 128), jnp.float32),
    )(wt, wt, wt, wt)


def _sc_gather(idx2d, table):
    mesh = plsc.VectorSubcoreMesh(core_axis_name="c", subcore_axis_name="s")

    @functools.partial(
        pl.kernel,
        mesh=mesh,
        out_type=jax.ShapeDtypeStruct((L, B, D), jnp.float32),
        scratch_types=[
            pltpu.VMEM((_GPC, _IDXW), jnp.int32),
            pltpu.VMEM((_GPC, _IDXW), jnp.int32),
            pltpu.VMEM((_CHUNK, D), jnp.float32),
            pltpu.SemaphoreType.DMA,
        ],
        compiler_params=pltpu.CompilerParams(use_tc_tiling_on_sc=False),
    )
    def k(idx_hbm, w_hbm, out_hbm, idx_v, idx_r, rows_v, sem):
        wid = lax.axis_index("s") * _NC + lax.axis_index("c")

        def body(i, carry):
            t = wid * _TASKS_PER_W + i
            l = t // _BBLK
            bb = t % _BBLK
            roff = l * (B // _IDXW) + bb * _GPC
            pltpu.sync_copy(idx_hbm.at[pl.ds(roff, _GPC)], idx_v)
            for j in range(_GPC):
                for c0 in range(0, _IDXW, 16):
                    v = idx_v[j, pl.ds(c0, 16)]
                    idx_r[j, pl.ds(c0, 16)] = (
                        ((v & (_SEG - 1)) << 2) | jax.lax.shift_right_logical(v, 18)
                    )
            for j in range(_GPC):
                pltpu.async_copy(
                    w_hbm.at[idx_r.at[j]],
                    rows_v.at[pl.ds(j * _IDXW, _IDXW)],
                    sem,
                )
            for j in range(_GPC):
                pltpu.make_async_copy(
                    w_hbm.at[idx_r.at[j]],
                    rows_v.at[pl.ds(j * _IDXW, _IDXW)],
                    sem,
                ).wait()
            pltpu.sync_copy(rows_v, out_hbm.at[l, pl.ds(bb * _CHUNK, _CHUNK)])
            return carry

        lax.fori_loop(0, _TASKS_PER_W, body, 0)

    return k(idx2d, table)


def _mask_body(wt_ref, out_ref, nz8_ref):
    i = pl.program_id(0)

    @pl.when(i == 0)
    def _():
        nz8_ref[...] = (wt_ref[...] != PAD).astype(jnp.int8)

    jcol = lax.broadcasted_iota(jnp.int32, (L, 1), 0)
    out_ref[...] = jnp.where(jcol <= i, nz8_ref[...], 0)[None]


def _mask_t8(words_t):
    return pl.pallas_call(
        _mask_body,
        grid=(L,),
        in_specs=[pl.BlockSpec((L, B), lambda i: (0, 0))],
        out_specs=pl.BlockSpec((1, L, B), lambda i: (i, 0, 0)),
        out_shape=jax.ShapeDtypeStruct((L, L, B), jnp.int8),
        scratch_shapes=[pltpu.VMEM((L, B), jnp.int8)],
    )(words_t)


def kernel(batch_words, W):
    words_t = batch_words.T
    idx2d = words_t.reshape(_TOTAL // _IDXW, _IDXW)
    w_lin = _tc_wprep(W.T).reshape(4 * _SEG, D)
    emb = _sc_gather(idx2d, w_lin).transpose(1, 0, 2)
    mask_t = _mask_t8(words_t) != 0
    masks = jnp.transpose(mask_t, (2, 0, 1))
    return emb, masks


# speedup vs baseline: 1.9555x; 1.0623x over previous
import functools

import jax
import jax.numpy as jnp
from jax import lax
from jax.experimental import pallas as pl
from jax.experimental.pallas import tpu as pltpu
from jax.experimental.pallas import tpu_sc as plsc

B = 4096
L = 200
D = 32
PAD = 0


_NC = 2
_NS = 16
_NW = _NC * _NS

_TOTAL = B * L
_PER_W = _TOTAL // _NW
_CHUNK = 1024
_N_CHUNKS = _PER_W // _CHUNK
_IDXW = 128
_GPC = _CHUNK // _IDXW


_BBLK = B // _CHUNK
_TASKS = L * _BBLK
_TASKS_PER_W = _TASKS // _NW

V = 1000000
_SEG = 1 << 18
_WBLK = 8192


def _wprep_body(w0, w1, w2, w3, out_ref):
    out_ref[:, 0:32] = w0[...].T
    out_ref[:, 32:64] = w1[...].T
    out_ref[:, 64:96] = w2[...].T
    out_ref[:, 96:128] = w3[...].T


def _tc_wprep(wt):
    nseg_blocks = _SEG // _WBLK
    max_blk = V // _WBLK
    specs = [
        pl.BlockSpec(
            (D, _WBLK),
            (lambda i, k=k: (0, jnp.minimum(k * nseg_blocks + i, max_blk))),
        )
        for k in range(4)
    ]
    return pl.pallas_call(
        _wprep_body,
        grid=(nseg_blocks,),
        in_specs=specs,
        out_specs=pl.BlockSpec((_WBLK, 128), lambda i: (i, 0)),
        out_shape=jax.ShapeDtypeStruct((_SEG, 128), jnp.float32),
    )(wt, wt, wt, wt)


def _sc_gather(idx2d, table):
    mesh = plsc.VectorSubcoreMesh(core_axis_name="c", subcore_axis_name="s")

    @functools.partial(
        pl.kernel,
        mesh=mesh,
        out_type=jax.ShapeDtypeStruct((L, B, D), jnp.float32),
        scratch_types=[
            pltpu.VMEM((_GPC, _IDXW), jnp.int32),
            pltpu.VMEM((_GPC, _IDXW), jnp.int32),
            pltpu.VMEM((_CHUNK, D), jnp.float32),
            pltpu.SemaphoreType.DMA,
        ],
        compiler_params=pltpu.CompilerParams(use_tc_tiling_on_sc=False),
    )
    def k(idx_hbm, w_hbm, out_hbm, idx_v, idx_r, rows_v, sem):
        wid = lax.axis_index("s") * _NC + lax.axis_index("c")

        def body(i, carry):
            t = wid * _TASKS_PER_W + i
            l = t // _BBLK
            bb = t % _BBLK
            roff = l * (B // _IDXW) + bb * _GPC
            pltpu.sync_copy(idx_hbm.at[pl.ds(roff, _GPC)], idx_v)
            for j in range(_GPC):
                for c0 in range(0, _IDXW, 16):
                    v = idx_v[j, pl.ds(c0, 16)]
                    idx_r[j, pl.ds(c0, 16)] = (
                        ((v & (_SEG - 1)) << 2) | jax.lax.shift_right_logical(v, 18)
                    )
            for j in range(_GPC):
                pltpu.async_copy(
                    w_hbm.at[idx_r.at[j]],
                    rows_v.at[pl.ds(j * _IDXW, _IDXW)],
                    sem,
                )
            for j in range(_GPC):
                pltpu.make_async_copy(
                    w_hbm.at[idx_r.at[j]],
                    rows_v.at[pl.ds(j * _IDXW, _IDXW)],
                    sem,
                ).wait()
            pltpu.sync_copy(rows_v, out_hbm.at[l, pl.ds(bb * _CHUNK, _CHUNK)])
            return carry

        lax.fori_loop(0, _TASKS_PER_W, body, 0)

    return k(idx2d, table)


def _mask_body(wt_ref, out_ref, nz8_ref):
    i = pl.program_id(0)

    @pl.when(i == 0)
    def _():
        nz8_ref[...] = (wt_ref[...] != PAD).astype(jnp.int8)

    jcol = lax.broadcasted_iota(jnp.int32, (L, 1), 0)
    out_ref[...] = jnp.where(jcol <= i, nz8_ref[...], 0)[None]


def _mask_t8(words_t):
    return pl.pallas_call(
        _mask_body,
        grid=(L,),
        in_specs=[pl.BlockSpec((L, B), lambda i: (0, 0))],
        out_specs=pl.BlockSpec((1, L, B), lambda i: (i, 0, 0)),
        out_shape=jax.ShapeDtypeStruct((L, L, B), jnp.int8),
        scratch_shapes=[pltpu.VMEM((L, B), jnp.int8)],
    )(words_t)


def kernel(batch_words, W):
    words_t = batch_words.T
    idx2d = words_t.reshape(_TOTAL // _IDXW, _IDXW)
    w_lin = _tc_wprep(W.T).reshape(4 * _SEG, D)
    emb = _sc_gather(idx2d, w_lin).transpose(1, 0, 2)
    mask_t = _mask_t8(words_t) != 0
    masks = jnp.transpose(mask_t, (2, 0, 1))
    return emb, masks


# speedup vs baseline: 1.9863x; 1.0157x over previous
import functools

import jax
import jax.numpy as jnp
from jax import lax
from jax.experimental import pallas as pl
from jax.experimental.pallas import tpu as pltpu
from jax.experimental.pallas import tpu_sc as plsc

B = 4096
L = 200
D = 32
PAD = 0


_NC = 2
_NS = 16
_NW = _NC * _NS

_TOTAL = B * L
_PER_W = _TOTAL // _NW
_CHUNK = 1024
_N_CHUNKS = _PER_W // _CHUNK
_IDXW = 128
_GPC = _CHUNK // _IDXW


_BBLK = B // _CHUNK
_TASKS = L * _BBLK
_TASKS_PER_W = _TASKS // _NW

V = 1000000
_SEG = 1 << 18
_WBLK = 8192


def _wprep_body(w0, w1, w2, w3, out_ref):
    out_ref[:, 0:32] = w0[...].T
    out_ref[:, 32:64] = w1[...].T
    out_ref[:, 64:96] = w2[...].T
    out_ref[:, 96:128] = w3[...].T


def _tc_wprep(wt):
    nseg_blocks = _SEG // _WBLK
    max_blk = V // _WBLK
    specs = [
        pl.BlockSpec(
            (D, _WBLK),
            (lambda i, k=k: (0, jnp.minimum(k * nseg_blocks + i, max_blk))),
        )
        for k in range(4)
    ]
    return pl.pallas_call(
        _wprep_body,
        grid=(nseg_blocks,),
        in_specs=specs,
        out_specs=pl.BlockSpec((_WBLK, 128), lambda i: (i, 0)),
        out_shape=jax.ShapeDtypeStruct((_SEG, 128), jnp.float32),
    )(wt, wt, wt, wt)


def _sc_gather(idx2d, table):
    mesh = plsc.VectorSubcoreMesh(core_axis_name="c", subcore_axis_name="s")

    @functools.partial(
        pl.kernel,
        mesh=mesh,
        out_type=jax.ShapeDtypeStruct((L, B, D), jnp.float32),
        scratch_types=[
            pltpu.VMEM((_GPC, _IDXW), jnp.int32),
            pltpu.VMEM((_GPC, _IDXW), jnp.int32),
            pltpu.VMEM((_CHUNK, D), jnp.float32),
            pltpu.SemaphoreType.DMA,
        ],
        compiler_params=pltpu.CompilerParams(use_tc_tiling_on_sc=False),
    )
    def k(idx_hbm, w_hbm, out_hbm, idx_v, idx_r, rows_v, sem):
        wid = lax.axis_index("s") * _NC + lax.axis_index("c")

        def body(i, carry):
            t = wid * _TASKS_PER_W + i
            l = t // _BBLK
            bb = t % _BBLK
            roff = l * (B // _IDXW) + bb * _GPC
            pltpu.sync_copy(idx_hbm.at[pl.ds(roff, _GPC)], idx_v)
            for j in range(_GPC):
                for c0 in range(0, _IDXW, 16):
                    v = idx_v[j, pl.ds(c0, 16)]
                    idx_r[j, pl.ds(c0, 16)] = (
                        ((v & (_SEG - 1)) << 2) | jax.lax.shift_right_logical(v, 18)
                    )
            for j in range(_GPC):
                pltpu.async_copy(
                    w_hbm.at[idx_r.at[j]],
                    rows_v.at[pl.ds(j * _IDXW, _IDXW)],
                    sem,
                )
            for j in range(_GPC):
                pltpu.make_async_copy(
                    w_hbm.at[idx_r.at[j]],
                    rows_v.at[pl.ds(j * _IDXW, _IDXW)],
                    sem,
                ).wait()
            pltpu.sync_copy(rows_v, out_hbm.at[l, pl.ds(bb * _CHUNK, _CHUNK)])
            return carry

        lax.fori_loop(0, _TASKS_PER_W, body, 0)

    return k(idx2d, table)


def _mask_body(wt_ref, out_ref, nz8_ref):
    i = pl.program_id(0)

    @pl.when(i == 0)
    def _():
        nz8_ref[...] = (wt_ref[...] != PAD).astype(jnp.int8)

    jcol = lax.broadcasted_iota(jnp.int32, (L, 1), 0)
    nz8 = nz8_ref[...]
    for t in range(_MBB):
        out_ref[t] = jnp.where(jcol <= i * _MBB + t, nz8, 0)


_MBB = 4


def _mask_t8(words_t):
    return pl.pallas_call(
        _mask_body,
        grid=(L // _MBB,),
        in_specs=[pl.BlockSpec((L, B), lambda i: (0, 0))],
        out_specs=pl.BlockSpec((_MBB, L, B), lambda i: (i, 0, 0)),
        out_shape=jax.ShapeDtypeStruct((L, L, B), jnp.int8),
        scratch_shapes=[pltpu.VMEM((L, B), jnp.int8)],
    )(words_t)


def kernel(batch_words, W):
    words_t = batch_words.T
    idx2d = words_t.reshape(_TOTAL // _IDXW, _IDXW)
    w_lin = _tc_wprep(W.T).reshape(4 * _SEG, D)
    emb = _sc_gather(idx2d, w_lin).transpose(1, 0, 2)
    mask_t = _mask_t8(words_t) != 0
    masks = jnp.transpose(mask_t, (2, 0, 1))
    return emb, masks
